# baseline (speedup 1.0000x reference)
import functools
import math

import jax
import jax.numpy as jnp
from jax import lax
from jax.experimental import pallas as pl
from jax.experimental.pallas import tpu as pltpu
from jax.experimental.pallas import tpu_sc as plsc

D_MODEL = 64
VOCAB = 1000
BATCH = 16384
HIST = 50
SCALE = math.sqrt(D_MODEL)

NC = 2
NS = 16
NW = NC * NS

B_TOTAL = BATCH * HIST
B_PER_W = B_TOTAL // NW
BATCH_PER_W = BATCH // NW
NB = 8
CHUNK = NB * HIST
NCHUNK = BATCH_PER_W // NB
NBUF = 2


def _scale_body(t_ref, o_ref):
    o_ref[...] = t_ref[...] * SCALE


def _scale_table(table):
    flat = table.reshape(VOCAB * D_MODEL // 128, 128)
    out = pl.pallas_call(
        _scale_body,
        out_shape=jax.ShapeDtypeStruct(flat.shape, jnp.float32),
    )(flat)
    return out.reshape(VOCAB, D_MODEL)


def _gather_body_for(batch_n):
    bpw = batch_n // NW
    b_per_w = bpw * HIST
    nchunk = bpw // NB

    def _gather_body(idx_hbm, table_hbm, out_hbm, idx_v, rows_v, table_sh,
                     *sems):
        gsems, wsems = sems[:NBUF], sems[NBUF:]
        cid = lax.axis_index("c")
        sid = lax.axis_index("s")
        wid = sid * NC + cid
        base = wid * b_per_w
        bbase = wid * bpw

        @pl.when(sid == 0)
        def _():
            pltpu.sync_copy(table_hbm, table_sh)

        pltpu.sync_copy(idx_hbm.at[pl.ds(base, b_per_w)], idx_v)
        plsc.subcore_barrier()

        def gather_desc(c, b):
            return pltpu.make_async_copy(
                table_sh.at[idx_v.at[pl.ds(c * CHUNK, CHUNK)]],
                rows_v.at[b],
                gsems[b],
            )

        def write_descs(c, b):
            return [
                pltpu.make_async_copy(
                    rows_v.at[b, pl.ds(n * HIST, HIST)],
                    out_hbm.at[bbase + c * NB + n],
                    wsems[b],
                )
                for n in range(NB)
            ]

        for b in range(NBUF):
            gather_desc(b, b).start()

        def body(g, _):
            c0 = g * NBUF
            for b in range(NBUF):
                gather_desc(c0 + b, b).wait()
                for d in write_descs(c0 + b, b):
                    d.start()
            for b in range(NBUF):
                @pl.when(c0 + b + NBUF < nchunk)
                def _():
                    for d in write_descs(c0 + b, b):
                        d.wait()
                    gather_desc(c0 + b + NBUF, b).start()
            return 0

        lax.fori_loop(0, nchunk // NBUF, body, 0)
        for b in range(NBUF):
            for d in write_descs(nchunk - NBUF + b, b):
                d.wait()

    return _gather_body


NSPLIT = 2


def kernel(x, table):
    idx = x.reshape(B_TOTAL).astype(jnp.int32)
    scaled = _scale_table(table)
    mesh = plsc.VectorSubcoreMesh(core_axis_name="c", subcore_axis_name="s")
    half = BATCH // NSPLIT
    gather = functools.partial(
        pl.kernel,
        mesh=mesh,
        out_type=jax.ShapeDtypeStruct((half, HIST, D_MODEL), jnp.float32),
        scratch_types=[
            pltpu.VMEM((half * HIST // NW,), jnp.int32),
            pltpu.VMEM((NBUF, CHUNK, D_MODEL), jnp.float32),
            pltpu.VMEM_SHARED((VOCAB, D_MODEL), jnp.float32),
        ] + [pltpu.SemaphoreType.DMA] * (2 * NBUF),
        compiler_params=pltpu.CompilerParams(use_tc_tiling_on_sc=False),
    )(_gather_body_for(half))
    n = half * HIST
    outs = [gather(idx[i * n:(i + 1) * n], scaled) for i in range(NSPLIT)]
    return jnp.concatenate(outs, axis=0)


# speedup vs baseline: 1.0749x; 1.0749x over previous
import functools
import math

import jax
import jax.numpy as jnp
from jax import lax
from jax.experimental import pallas as pl
from jax.experimental.pallas import tpu as pltpu
from jax.experimental.pallas import tpu_sc as plsc

D_MODEL = 64
VOCAB = 1000
BATCH = 16384
HIST = 50
SCALE = math.sqrt(D_MODEL)

NC = 2
NS = 16
NW = NC * NS

B_TOTAL = BATCH * HIST
B_PER_W = B_TOTAL // NW
BATCH_PER_W = BATCH // NW
NB = 8
CHUNK = NB * HIST
NCHUNK = BATCH_PER_W // NB
NBUF = 2


def _scale_body(t_ref, o_ref):
    o_ref[...] = t_ref[...] * SCALE


def _scale_table(table):
    flat = table.reshape(VOCAB * D_MODEL // 128, 128)
    out = pl.pallas_call(
        _scale_body,
        out_shape=jax.ShapeDtypeStruct(flat.shape, jnp.float32),
    )(flat)
    return out.reshape(VOCAB, D_MODEL)


def _gather_body(idx_hbm, table_hbm, out_hbm, idx_v, rows_v, table_sh, *sems):
    gsems, wsems = sems[:NBUF], sems[NBUF:]
    cid = lax.axis_index("c")
    sid = lax.axis_index("s")
    wid = sid * NC + cid
    base = wid * B_PER_W
    bbase = wid * BATCH_PER_W

    @pl.when(sid == 0)
    def _():
        pltpu.sync_copy(table_hbm, table_sh)

    pltpu.sync_copy(idx_hbm.at[pl.ds(base, B_PER_W)], idx_v)
    plsc.subcore_barrier()

    def gather_desc(c, b):
        return pltpu.make_async_copy(
            table_sh.at[idx_v.at[pl.ds(c * CHUNK, CHUNK)]],
            rows_v.at[b],
            gsems[b],
        )

    def write_descs(c, b):
        return [
            pltpu.make_async_copy(
                rows_v.at[b, pl.ds(n * HIST, HIST)],
                out_hbm.at[bbase + c * NB + n],
                wsems[b],
            )
            for n in range(NB)
        ]

    for b in range(NBUF):
        gather_desc(b, b).start()

    def body(g, _):
        c0 = g * NBUF
        for b in range(NBUF):
            gather_desc(c0 + b, b).wait()
            for d in write_descs(c0 + b, b):
                d.start()
        for b in range(NBUF):
            @pl.when(c0 + b + NBUF < NCHUNK)
            def _():
                for d in write_descs(c0 + b, b):
                    d.wait()
                gather_desc(c0 + b + NBUF, b).start()
        return 0

    lax.fori_loop(0, NCHUNK // NBUF, body, 0)
    for b in range(NBUF):
        for d in write_descs(NCHUNK - NBUF + b, b):
            d.wait()


def kernel(x, table):
    idx = x.reshape(B_TOTAL).astype(jnp.int32)
    scaled = _scale_table(table)
    mesh = plsc.VectorSubcoreMesh(core_axis_name="c", subcore_axis_name="s")
    gather = functools.partial(
        pl.kernel,
        mesh=mesh,
        out_type=jax.ShapeDtypeStruct((BATCH, HIST, D_MODEL), jnp.float32),
        scratch_types=[
            pltpu.VMEM((B_PER_W,), jnp.int32),
            pltpu.VMEM((NBUF, CHUNK, D_MODEL), jnp.float32),
            pltpu.VMEM_SHARED((VOCAB, D_MODEL), jnp.float32),
        ] + [pltpu.SemaphoreType.DMA] * (2 * NBUF),
        compiler_params=pltpu.CompilerParams(use_tc_tiling_on_sc=False),
    )(_gather_body)
    return gather(idx, scaled)
